# baseline (speedup 1.0000x reference)
import functools

import jax
import jax.numpy as jnp
from jax import lax
from jax.experimental import pallas as pl
from jax.experimental.pallas import tpu as pltpu
from jax.experimental.pallas import tpu_sc as plsc

B = 16384
F = 26
FIELD = 38462
OFFS = [f * FIELD for f in range(F)]
NC = 2
NS = 16
NW = NC * NS
BPW = B // NW
IPW = BPW * F
GROW = 128
NROW = IPW // GROW
CHUNKS = BPW // 16


def _body(xt_hbm, wf_hbm, bias_hbm, out_hbm,
          xv, idxv, vals, outv, bv, sem0, sem1):
    wid = lax.axis_index("s") * NC + lax.axis_index("c")
    base = wid * BPW
    sems = (sem0, sem1)

    pltpu.sync_copy(xt_hbm.at[:, pl.ds(base, BPW)], xv)
    pltpu.sync_copy(bias_hbm, bv)

    HF = F // 2
    HI = HF * BPW
    bias_v = bv[...]

    def build_half(h):
        f0 = h * HF

        def bh(c, _):
            s = c * 16
            for f in range(f0, f0 + HF):
                idxv[pl.ds(f * BPW + s, 16)] = (
                    xv[f, pl.ds(s, 16)] + OFFS[f])
            return _
        lax.fori_loop(0, CHUNKS, bh, None)

    def fire_half(h, sem):
        pltpu.async_copy(
            wf_hbm.at[idxv.at[pl.ds(h * HI, HI)]],
            vals.at[pl.ds(h * HI, HI)],
            sem)

    def drain_half(h, sem):
        pltpu.make_async_copy(
            wf_hbm.at[pl.ds(0, HI)],
            vals.at[pl.ds(h * HI, HI)],
            sem).wait()

    build_half(0)
    fire_half(0, sems[0])
    build_half(1)
    fire_half(1, sems[1])

    drain_half(0, sems[0])

    def acc0(c, _):
        s = c * 16
        acc = bias_v
        for f in range(HF):
            acc = acc + vals[pl.ds(f * BPW + s, 16)]
        outv[pl.ds(s, 16)] = acc
        return _
    lax.fori_loop(0, CHUNKS, acc0, None)

    drain_half(1, sems[1])

    def acc1(c, _):
        s = c * 16
        acc = outv[pl.ds(s, 16)]
        for f in range(HF, F):
            acc = acc + vals[pl.ds(f * BPW + s, 16)]
        outv[pl.ds(s, 16)] = 1.0 / (1.0 + jnp.exp(-acc))
        return _
    lax.fori_loop(0, CHUNKS, acc1, None)

    pltpu.sync_copy(outv, out_hbm.at[pl.ds(base, BPW)])


def kernel(x, W, bias):
    xt = x.astype(jnp.int32).T
    pad = 1000448 - W.shape[0]
    wf = jnp.pad(W.T, ((0, 0), (0, pad))).reshape(-1)
    bias16 = jnp.broadcast_to(bias.astype(jnp.float32), (16,))

    mesh = plsc.VectorSubcoreMesh(core_axis_name="c", subcore_axis_name="s")
    run = functools.partial(
        pl.kernel,
        mesh=mesh,
        out_type=jax.ShapeDtypeStruct((B,), jnp.float32),
        scratch_types=[
            pltpu.VMEM((F, BPW), jnp.int32),
            pltpu.VMEM((IPW,), jnp.int32),
            pltpu.VMEM((IPW,), jnp.float32),
            pltpu.VMEM((BPW,), jnp.float32),
            pltpu.VMEM((16,), jnp.float32),
            pltpu.SemaphoreType.DMA,
            pltpu.SemaphoreType.DMA,
        ],
    )(_body)
    return run(xt, wf, bias16)


# speedup vs baseline: 1.0086x; 1.0086x over previous
import functools

import jax
import jax.numpy as jnp
from jax import lax
from jax.experimental import pallas as pl
from jax.experimental.pallas import tpu as pltpu
from jax.experimental.pallas import tpu_sc as plsc

B = 16384
F = 26
FIELD = 38462
OFFS = [f * FIELD for f in range(F)]
NC = 2
NS = 16
NW = NC * NS
BPW = B // NW
IPW = BPW * F
CHUNKS = BPW // 16
HF = F // 2
HI = HF * BPW
WPAD = 1000448


def _build_body(xt_hbm, idx_hbm, xv, idxv):
    wid = lax.axis_index("s") * NC + lax.axis_index("c")
    base = wid * BPW

    pltpu.sync_copy(xt_hbm.at[:, pl.ds(base, BPW)], xv)

    def build(c, _):
        s = c * 16
        for f in range(F):
            idxv[pl.ds(f * BPW + s, 16)] = xv[f, pl.ds(s, 16)] + OFFS[f]
        return _
    lax.fori_loop(0, CHUNKS, build, None)

    pltpu.sync_copy(idxv, idx_hbm.at[pl.ds(wid * IPW, IPW)])


def _gather_body(idx_hbm, wf_hbm, bias_hbm, out_hbm,
                 idxv, vals, outv, bv, sem0, sem1):
    wid = lax.axis_index("s") * NC + lax.axis_index("c")
    base = wid * BPW
    sems = (sem0, sem1)

    pltpu.sync_copy(idx_hbm.at[pl.ds(wid * IPW, IPW)], idxv)
    pltpu.sync_copy(bias_hbm, bv)
    bias_v = bv[...]

    def fire_half(h, sem):
        pltpu.async_copy(
            wf_hbm.at[idxv.at[pl.ds(h * HI, HI)]],
            vals.at[pl.ds(h * HI, HI)],
            sem)

    def drain_half(h, sem):
        pltpu.make_async_copy(
            wf_hbm.at[pl.ds(0, HI)],
            vals.at[pl.ds(h * HI, HI)],
            sem).wait()

    fire_half(0, sems[0])
    fire_half(1, sems[1])

    drain_half(0, sems[0])

    def acc0(c, _):
        s = c * 16
        acc = bias_v
        for f in range(HF):
            acc = acc + vals[pl.ds(f * BPW + s, 16)]
        outv[pl.ds(s, 16)] = acc
        return _
    lax.fori_loop(0, CHUNKS, acc0, None)

    drain_half(1, sems[1])

    def acc1(c, _):
        s = c * 16
        acc = outv[pl.ds(s, 16)]
        for f in range(HF, F):
            acc = acc + vals[pl.ds(f * BPW + s, 16)]
        outv[pl.ds(s, 16)] = 1.0 / (1.0 + jnp.exp(-acc))
        return _
    lax.fori_loop(0, CHUNKS, acc1, None)

    pltpu.sync_copy(outv, out_hbm.at[pl.ds(base, BPW)])


def kernel(x, W, bias):
    xt = x.astype(jnp.int32).T
    wf = jnp.pad(W, ((0, WPAD - W.shape[0]), (0, 0))).reshape(-1)
    bias16 = jnp.broadcast_to(bias.astype(jnp.float32), (16,))

    mesh = plsc.VectorSubcoreMesh(core_axis_name="c", subcore_axis_name="s")

    build = functools.partial(
        pl.kernel,
        mesh=mesh,
        out_type=jax.ShapeDtypeStruct((B * F,), jnp.int32),
        scratch_types=[
            pltpu.VMEM((F, BPW), jnp.int32),
            pltpu.VMEM((IPW,), jnp.int32),
        ],
    )(_build_body)

    gather = functools.partial(
        pl.kernel,
        mesh=mesh,
        out_type=jax.ShapeDtypeStruct((B,), jnp.float32),
        scratch_types=[
            pltpu.VMEM((IPW,), jnp.int32),
            pltpu.VMEM((IPW,), jnp.float32),
            pltpu.VMEM((BPW,), jnp.float32),
            pltpu.VMEM((16,), jnp.float32),
            pltpu.SemaphoreType.DMA,
            pltpu.SemaphoreType.DMA,
        ],
    )(_gather_body)

    return gather(build(xt), wf, bias16)


# speedup vs baseline: 1.0154x; 1.0067x over previous
import functools

import jax
import jax.numpy as jnp
from jax import lax
from jax.experimental import pallas as pl
from jax.experimental.pallas import tpu as pltpu
from jax.experimental.pallas import tpu_sc as plsc

B = 16384
F = 26
FIELD = 38462
OFFS = [f * FIELD for f in range(F)]
NC = 2
NS = 16
NW = NC * NS
BPW = B // NW
IPW = BPW * F
CHUNKS = BPW // 16
HF = F // 2
HI = HF * BPW
WPAD = 1000448


def _build_body(xt_hbm, idx_hbm, xv, idxv, semo):
    wid = lax.axis_index("s") * NC + lax.axis_index("c")
    base = wid * BPW

    pltpu.sync_copy(xt_hbm.at[:, pl.ds(base, BPW)], xv)

    def build_half(h):
        def bh(c, _):
            s = c * 16
            for f in range(h * HF, (h + 1) * HF):
                idxv[pl.ds(f * BPW + s, 16)] = (
                    xv[f, pl.ds(s, 16)] + OFFS[f])
            return _
        lax.fori_loop(0, CHUNKS, bh, None)

    build_half(0)
    pltpu.async_copy(
        idxv.at[pl.ds(0, HI)], idx_hbm.at[pl.ds(wid * IPW, HI)], semo)
    build_half(1)
    pltpu.async_copy(
        idxv.at[pl.ds(HI, HI)], idx_hbm.at[pl.ds(wid * IPW + HI, HI)], semo)
    pltpu.make_async_copy(
        idxv, idx_hbm.at[pl.ds(wid * IPW, IPW)], semo).wait()


def _gather_body(idx_hbm, wf_hbm, bias_hbm, out_hbm,
                 idxv, vals, outv, bv, sem0, sem1):
    wid = lax.axis_index("s") * NC + lax.axis_index("c")
    base = wid * BPW
    sems = (sem0, sem1)

    bv[pl.ds(0, 16)] = jnp.zeros((16,), jnp.float32)
    pltpu.sync_copy(idx_hbm.at[pl.ds(wid * IPW, IPW)], idxv)
    pltpu.sync_copy(bias_hbm, bv.at[pl.ds(0, 1)])
    bias_v = jnp.broadcast_to(jnp.sum(bv[pl.ds(0, 16)]), (16,))

    def fire_half(h, sem):
        pltpu.async_copy(
            wf_hbm.at[idxv.at[pl.ds(h * HI, HI)]],
            vals.at[pl.ds(h * HI, HI)],
            sem)

    def drain_half(h, sem):
        pltpu.make_async_copy(
            wf_hbm.at[pl.ds(0, HI)],
            vals.at[pl.ds(h * HI, HI)],
            sem).wait()

    fire_half(0, sems[0])
    fire_half(1, sems[1])

    drain_half(0, sems[0])

    def acc0(c, _):
        s = c * 16
        acc = bias_v
        for f in range(HF):
            acc = acc + vals[pl.ds(f * BPW + s, 16)]
        outv[pl.ds(s, 16)] = acc
        return _
    lax.fori_loop(0, CHUNKS, acc0, None)

    drain_half(1, sems[1])

    def acc1(c, _):
        s = c * 16
        acc = outv[pl.ds(s, 16)]
        for f in range(HF, F):
            acc = acc + vals[pl.ds(f * BPW + s, 16)]
        outv[pl.ds(s, 16)] = 1.0 / (1.0 + jnp.exp(-acc))
        return _
    lax.fori_loop(0, CHUNKS, acc1, None)

    pltpu.sync_copy(outv, out_hbm.at[pl.ds(base, BPW)])


def kernel(x, W, bias):
    xt = x.astype(jnp.int32).T
    wf = jnp.pad(W, ((0, WPAD - W.shape[0]), (0, 0))).reshape(-1)

    mesh = plsc.VectorSubcoreMesh(core_axis_name="c", subcore_axis_name="s")

    build = functools.partial(
        pl.kernel,
        mesh=mesh,
        out_type=jax.ShapeDtypeStruct((B * F,), jnp.int32),
        scratch_types=[
            pltpu.VMEM((F, BPW), jnp.int32),
            pltpu.VMEM((IPW,), jnp.int32),
            pltpu.SemaphoreType.DMA,
        ],
    )(_build_body)

    gather = functools.partial(
        pl.kernel,
        mesh=mesh,
        out_type=jax.ShapeDtypeStruct((B,), jnp.float32),
        compiler_params=pltpu.CompilerParams(needs_layout_passes=False),
        scratch_types=[
            pltpu.VMEM((IPW,), jnp.int32),
            pltpu.VMEM((IPW,), jnp.float32),
            pltpu.VMEM((BPW,), jnp.float32),
            pltpu.VMEM((16,), jnp.float32),
            pltpu.SemaphoreType.DMA,
            pltpu.SemaphoreType.DMA,
        ],
    )(_gather_body)

    return gather(build(xt), wf, bias.astype(jnp.float32))
